# baseline (speedup 1.0000x reference)
import dataclasses
import functools

import jax
import jax.numpy as jnp
from jax import lax
from jax.experimental import pallas as pl
from jax.experimental.pallas import tpu as pltpu
from jax.experimental.pallas import tpu_sc as plsc

NC = 2
NS = 16
NW = NC * NS
LANES = 16
CHUNK = 120
NROWS = 3
NMETA = 4
ZROWS = 80


def _sc_gather_scatter(x, src, dst, w, n_pad, ept):
    d = x.shape[1]
    nchunks = ept // CHUNK
    rows_per_sub = n_pad // NS
    zcopies = rows_per_sub // ZROWS
    cycle = NROWS * NMETA
    assert nchunks % cycle == 0 and nchunks >= 2 * cycle
    mesh = plsc.VectorSubcoreMesh(core_axis_name="c", subcore_axis_name="s")
    cp = pltpu.CompilerParams()
    if "needs_layout_passes" in pltpu.CompilerParams.__dataclass_fields__:
        cp = dataclasses.replace(cp, needs_layout_passes=False)

    scratch = (
        [pltpu.VMEM((CHUNK, 128), jnp.float32) for _ in range(NROWS)]
        + [pltpu.VMEM((CHUNK,), jnp.int32) for _ in range(NMETA)]
        + [pltpu.VMEM((CHUNK,), jnp.int32) for _ in range(NMETA)]
        + [pltpu.VMEM((CHUNK,), jnp.float32) for _ in range(NMETA)]
        + [pltpu.VMEM_SHARED((n_pad, 128), jnp.float32)]
        + [pltpu.SemaphoreType.DMA for _ in range(NROWS + NMETA)]
    )

    @functools.partial(
        pl.kernel,
        compiler_params=cp,
        out_type=jax.ShapeDtypeStruct((NC, n_pad, d), jnp.float32),
        mesh=mesh,
        scratch_types=scratch,
    )
    def sc_kernel(x_hbm, src_hbm, dst_hbm, w_hbm, out_hbm, *refs):
        rows = refs[:NROWS]
        srcs = refs[NROWS:NROWS + NMETA]
        dsts = refs[NROWS + NMETA:NROWS + 2 * NMETA]
        ws = refs[NROWS + 2 * NMETA:NROWS + 3 * NMETA]
        acc_sh = refs[NROWS + 3 * NMETA]
        semg = refs[NROWS + 3 * NMETA + 1:NROWS + 3 * NMETA + 1 + NROWS]
        semm = refs[NROWS + 3 * NMETA + 1 + NROWS:]

        cid = lax.axis_index("c")
        sid = lax.axis_index("s")
        wid = sid * NC + cid

        def meta_load(c, j):
            sl = pl.ds(pl.multiple_of(wid * ept + c * CHUNK, 8), CHUNK)
            pltpu.async_copy(src_hbm.at[sl], srcs[j], semm[j])
            pltpu.async_copy(dst_hbm.at[sl], dsts[j], semm[j])
            pltpu.async_copy(w_hbm.at[sl], ws[j], semm[j])

        def meta_wait(c, j):
            sl = pl.ds(pl.multiple_of(wid * ept + c * CHUNK, 8), CHUNK)
            pltpu.make_async_copy(src_hbm.at[sl], srcs[j], semm[j]).wait()
            pltpu.make_async_copy(dst_hbm.at[sl], dsts[j], semm[j]).wait()
            pltpu.make_async_copy(w_hbm.at[sl], ws[j], semm[j]).wait()

        def scale_rows(buf, wb):
            @plsc.parallel_loop(0, CHUNK, unroll=8)
            def _(r):
                wk = plsc.load_gather(wb, [jnp.full((LANES,), r, jnp.int32)])
                for c2 in range(d // LANES):
                    sl = pl.ds(c2 * LANES, LANES)
                    buf[r, sl] = buf[r, sl] * wk

        def substep(c, j, do_gather, do_meta):
            rb, rs = rows[j % NROWS], semg[j % NROWS]
            jm = j % NMETA
            pltpu.make_async_copy(x_hbm.at[srcs[jm]], rb, rs).wait()
            scale_rows(rb, ws[jm])
            pltpu.sync_copy(rb, acc_sh.at[dsts[jm]], add=True)
            if do_gather:
                j2 = (j + 2) % NMETA
                meta_wait(c + 2, j2)
                pltpu.async_copy(x_hbm.at[srcs[j2]],
                                 rows[(j + 2) % NROWS], semg[(j + 2) % NROWS])
            if do_meta:
                meta_load(c + 3, (j + 3) % NMETA)

        @pl.loop(0, ZROWS)
        def _(r):
            for c2 in range(d // LANES):
                rows[0][r, pl.ds(c2 * LANES, LANES)] = jnp.zeros(
                    (LANES,), jnp.float32)

        zsrc = rows[0].at[pl.ds(0, ZROWS)]

        @pl.loop(0, zcopies)
        def _(j):
            pltpu.sync_copy(
                zsrc,
                acc_sh.at[pl.ds(
                    pl.multiple_of(sid * rows_per_sub + j * ZROWS, 8),
                    ZROWS)])

        plsc.subcore_barrier()

        meta_load(0, 0)
        meta_load(1, 1)
        meta_load(2, 2)
        meta_wait(0, 0)
        pltpu.async_copy(x_hbm.at[srcs[0]], rows[0], semg[0])
        meta_wait(1, 1)
        pltpu.async_copy(x_hbm.at[srcs[1]], rows[1], semg[1])

        @pl.loop(0, nchunks // cycle - 1)
        def _(h):
            c0 = h * cycle
            for jj in range(cycle):
                substep(c0 + jj, jj, True, True)

        for jj in range(cycle):
            c = nchunks - cycle + jj
            substep(c, jj, c + 2 < nchunks, c + 3 < nchunks)

        plsc.subcore_barrier()

        @pl.loop(0, zcopies)
        def _(j):
            rb = pl.multiple_of(sid * rows_per_sub + j * ZROWS, 8)
            pltpu.sync_copy(acc_sh.at[pl.ds(rb, ZROWS)],
                            out_hbm.at[cid, pl.ds(rb, ZROWS)])

    return sc_kernel(x, src, dst, w)


def _tc_linear_pool(partials, batch3, W, b2, g, blk):
    n_pad, d = partials.shape[1], partials.shape[2]
    nblk = n_pad // blk

    def body(p_ref, batch_ref, w_ref, b_ref, g_ref):
        i = pl.program_id(0)
        agg = p_ref[0] + p_ref[1]
        z = jnp.dot(agg, w_ref[...], precision=lax.Precision.HIGHEST,
                    preferred_element_type=jnp.float32)
        z = jnp.maximum(z + b_ref[...], 0.0)
        bvec = batch_ref[0, 0, :]
        gid = lax.broadcasted_iota(jnp.int32, (g, blk), 0)
        onehot = (bvec[None, :] == gid).astype(jnp.float32)
        contrib = jnp.dot(onehot, z, precision=lax.Precision.HIGHEST,
                          preferred_element_type=jnp.float32)

        @pl.when(i == 0)
        def _():
            g_ref[...] = jnp.zeros_like(g_ref)

        g_ref[...] += contrib

    return pl.pallas_call(
        body,
        grid=(nblk,),
        in_specs=[
            pl.BlockSpec((NC, blk, d), lambda i: (0, i, 0)),
            pl.BlockSpec((1, 1, blk), lambda i: (i, 0, 0)),
            pl.BlockSpec((d, d), lambda i: (0, 0)),
            pl.BlockSpec((1, d), lambda i: (0, 0)),
        ],
        out_specs=pl.BlockSpec((g, d), lambda i: (0, 0)),
        out_shape=jax.ShapeDtypeStruct((g, d), jnp.float32),
    )(partials, batch3, W, b2)


def kernel(batch, x, edge_index, edge_weight, W, b):
    n, d = x.shape
    e = edge_index.shape[1]
    g = 128

    cyc = CHUNK * NROWS * NMETA
    ept = -(-e // NW)
    ept = max(-(-ept // cyc) * cyc, 2 * cyc)
    e_pad = NW * ept

    blk = 1024
    n_pad = -(-n // blk) * blk

    src = edge_index[0]
    dst = edge_index[1]
    pad_e = e_pad - e
    if pad_e:
        src = jnp.concatenate([src, jnp.zeros((pad_e,), jnp.int32)])
        dst = jnp.concatenate([dst, jnp.full((pad_e,), n, jnp.int32)])
        edge_weight = jnp.concatenate(
            [edge_weight, jnp.zeros((pad_e,), jnp.float32)])


    partials = _sc_gather_scatter(x, src, dst, edge_weight, n_pad, ept)

    batch_pad = jnp.concatenate(
        [batch, jnp.full((n_pad - n,), g, jnp.int32)]).reshape(
            n_pad // blk, 1, blk)

    return _tc_linear_pool(partials, batch_pad, W, b.reshape(1, d), g, blk)


# speedup vs baseline: 1.0465x; 1.0465x over previous
import dataclasses
import functools

import jax
import jax.numpy as jnp
from jax import lax
from jax.experimental import pallas as pl
from jax.experimental.pallas import tpu as pltpu
from jax.experimental.pallas import tpu_sc as plsc

NC = 2
NS = 16
NW = NC * NS
LANES = 16
CHUNK = 120
NROWS = 3
NMETA = 4
ZROWS = 80


def _sc_gather_scatter(x, src, dst, w, n_pad, ept):
    d = x.shape[1]
    nchunks = ept // CHUNK
    rows_per_sub = n_pad // NS
    zcopies = rows_per_sub // ZROWS
    cycle = NROWS * NMETA
    assert nchunks % cycle == 0 and nchunks >= 2 * cycle
    mesh = plsc.VectorSubcoreMesh(core_axis_name="c", subcore_axis_name="s")
    cp = pltpu.CompilerParams()
    if "needs_layout_passes" in pltpu.CompilerParams.__dataclass_fields__:
        cp = dataclasses.replace(cp, needs_layout_passes=False)

    scratch = (
        [pltpu.VMEM((CHUNK, 128), jnp.float32) for _ in range(NROWS)]
        + [pltpu.VMEM((CHUNK,), jnp.int32) for _ in range(NMETA)]
        + [pltpu.VMEM((CHUNK,), jnp.int32) for _ in range(NMETA)]
        + [pltpu.VMEM((CHUNK,), jnp.float32) for _ in range(NMETA)]
        + [pltpu.VMEM_SHARED((n_pad, 128), jnp.float32)]
        + [pltpu.SemaphoreType.DMA for _ in range(2 * NROWS + NMETA)]
    )

    @functools.partial(
        pl.kernel,
        compiler_params=cp,
        out_type=jax.ShapeDtypeStruct((NC, n_pad, d), jnp.float32),
        mesh=mesh,
        scratch_types=scratch,
    )
    def sc_kernel(x_hbm, src_hbm, dst_hbm, w_hbm, out_hbm, *refs):
        rows = refs[:NROWS]
        srcs = refs[NROWS:NROWS + NMETA]
        dsts = refs[NROWS + NMETA:NROWS + 2 * NMETA]
        ws = refs[NROWS + 2 * NMETA:NROWS + 3 * NMETA]
        acc_sh = refs[NROWS + 3 * NMETA]
        semg = refs[NROWS + 3 * NMETA + 1:NROWS + 3 * NMETA + 1 + NROWS]
        semm = refs[NROWS + 3 * NMETA + 1 + NROWS:
                    NROWS + 3 * NMETA + 1 + NROWS + NMETA]
        semsc = refs[NROWS + 3 * NMETA + 1 + NROWS + NMETA:]

        cid = lax.axis_index("c")
        sid = lax.axis_index("s")
        wid = sid * NC + cid

        def meta_load(c, j):
            sl = pl.ds(pl.multiple_of(wid * ept + c * CHUNK, 8), CHUNK)
            pltpu.async_copy(src_hbm.at[sl], srcs[j], semm[j])
            pltpu.async_copy(dst_hbm.at[sl], dsts[j], semm[j])
            pltpu.async_copy(w_hbm.at[sl], ws[j], semm[j])

        def meta_wait(c, j):
            sl = pl.ds(pl.multiple_of(wid * ept + c * CHUNK, 8), CHUNK)
            pltpu.make_async_copy(src_hbm.at[sl], srcs[j], semm[j]).wait()
            pltpu.make_async_copy(dst_hbm.at[sl], dsts[j], semm[j]).wait()
            pltpu.make_async_copy(w_hbm.at[sl], ws[j], semm[j]).wait()

        def scale_rows(buf, wb):
            @plsc.parallel_loop(0, CHUNK, unroll=4)
            def _(r):
                wk = plsc.load_gather(wb, [jnp.full((LANES,), r, jnp.int32)])
                for c2 in range(d // LANES):
                    sl = pl.ds(c2 * LANES, LANES)
                    buf[r, sl] = buf[r, sl] * wk

        def scatter_wait(jp):
            pltpu.make_async_copy(rows[jp % NROWS],
                                  acc_sh.at[dsts[jp % NMETA]],
                                  semsc[jp % NROWS]).wait()

        def substep(c, j, do_gather, do_meta, wait_prev=True):
            rb, rs = rows[j % NROWS], semg[j % NROWS]
            jm = j % NMETA
            pltpu.make_async_copy(x_hbm.at[srcs[jm]], rb, rs).wait()
            scale_rows(rb, ws[jm])
            pltpu.async_copy(rb, acc_sh.at[dsts[jm]], semsc[j % NROWS],
                             add=True)
            if do_gather:
                j2 = (j + 2) % NMETA
                meta_wait(c + 2, j2)
                if wait_prev:
                    scatter_wait((j - 1) % cycle)
                pltpu.async_copy(x_hbm.at[srcs[j2]],
                                 rows[(j + 2) % NROWS], semg[(j + 2) % NROWS])
            if do_meta:
                meta_load(c + 3, (j + 3) % NMETA)

        @pl.loop(0, ZROWS)
        def _(r):
            for c2 in range(d // LANES):
                rows[0][r, pl.ds(c2 * LANES, LANES)] = jnp.zeros(
                    (LANES,), jnp.float32)

        zsrc = rows[0].at[pl.ds(0, ZROWS)]

        @pl.loop(0, zcopies)
        def _(j):
            pltpu.sync_copy(
                zsrc,
                acc_sh.at[pl.ds(
                    pl.multiple_of(sid * rows_per_sub + j * ZROWS, 8),
                    ZROWS)])

        plsc.subcore_barrier()

        meta_load(0, 0)
        meta_load(1, 1)
        meta_load(2, 2)
        meta_wait(0, 0)
        pltpu.async_copy(x_hbm.at[srcs[0]], rows[0], semg[0])
        meta_wait(1, 1)
        pltpu.async_copy(x_hbm.at[srcs[1]], rows[1], semg[1])

        for jj in range(cycle):
            substep(jj, jj, True, True, wait_prev=(jj != 0))

        @pl.loop(0, nchunks // cycle - 2)
        def _(h):
            c0 = (h + 1) * cycle
            for jj in range(cycle):
                substep(c0 + jj, jj, True, True)

        for jj in range(cycle):
            c = nchunks - cycle + jj
            substep(c, jj, c + 2 < nchunks, c + 3 < nchunks)

        scatter_wait((nchunks - 3) % cycle)
        scatter_wait((nchunks - 2) % cycle)
        scatter_wait((nchunks - 1) % cycle)

        plsc.subcore_barrier()

        @pl.loop(0, zcopies)
        def _(j):
            rb = pl.multiple_of(sid * rows_per_sub + j * ZROWS, 8)
            pltpu.sync_copy(acc_sh.at[pl.ds(rb, ZROWS)],
                            out_hbm.at[cid, pl.ds(rb, ZROWS)])

    return sc_kernel(x, src, dst, w)


def _tc_linear_pool(partials, batch3, W, b2, g, blk):
    n_pad, d = partials.shape[1], partials.shape[2]
    nblk = n_pad // blk

    def body(p_ref, batch_ref, w_ref, b_ref, g_ref):
        i = pl.program_id(0)
        agg = p_ref[0] + p_ref[1]
        z = jnp.dot(agg, w_ref[...], precision=lax.Precision.HIGHEST,
                    preferred_element_type=jnp.float32)
        z = jnp.maximum(z + b_ref[...], 0.0)
        bvec = batch_ref[0, 0, :]
        gid = lax.broadcasted_iota(jnp.int32, (g, blk), 0)
        onehot = (bvec[None, :] == gid).astype(jnp.float32)
        contrib = jnp.dot(onehot, z, precision=lax.Precision.HIGHEST,
                          preferred_element_type=jnp.float32)

        @pl.when(i == 0)
        def _():
            g_ref[...] = jnp.zeros_like(g_ref)

        g_ref[...] += contrib

    return pl.pallas_call(
        body,
        grid=(nblk,),
        in_specs=[
            pl.BlockSpec((NC, blk, d), lambda i: (0, i, 0)),
            pl.BlockSpec((1, 1, blk), lambda i: (i, 0, 0)),
            pl.BlockSpec((d, d), lambda i: (0, 0)),
            pl.BlockSpec((1, d), lambda i: (0, 0)),
        ],
        out_specs=pl.BlockSpec((g, d), lambda i: (0, 0)),
        out_shape=jax.ShapeDtypeStruct((g, d), jnp.float32),
    )(partials, batch3, W, b2)


def kernel(batch, x, edge_index, edge_weight, W, b):
    n, d = x.shape
    e = edge_index.shape[1]
    g = 128

    cyc = CHUNK * NROWS * NMETA
    ept = -(-e // NW)
    ept = max(-(-ept // cyc) * cyc, 2 * cyc)
    e_pad = NW * ept

    blk = 1024
    n_pad = -(-n // blk) * blk

    src = edge_index[0]
    dst = edge_index[1]
    pad_e = e_pad - e
    if pad_e:
        src = jnp.concatenate([src, jnp.zeros((pad_e,), jnp.int32)])
        dst = jnp.concatenate([dst, jnp.full((pad_e,), n, jnp.int32)])
        edge_weight = jnp.concatenate(
            [edge_weight, jnp.zeros((pad_e,), jnp.float32)])


    partials = _sc_gather_scatter(x, src, dst, edge_weight, n_pad, ept)

    batch_pad = jnp.concatenate(
        [batch, jnp.full((n_pad - n,), g, jnp.int32)]).reshape(
            n_pad // blk, 1, blk)

    return _tc_linear_pool(partials, batch_pad, W, b.reshape(1, d), g, blk)
